# Initial kernel scaffold; baseline (speedup 1.0000x reference)
import jax
import jax.numpy as jnp
from jax.experimental import pallas as pl


def kernel(X, raw_emb_tables):
    raise NotImplementedError("write your pallas kernel here")


# speedup vs baseline: 12.2098x; 12.2098x over previous
import jax
import jax.numpy as jnp
from jax import lax
from jax.experimental import pallas as pl
from jax.experimental.pallas import tpu as pltpu
from jax.experimental.pallas import tpu_sc as plsc

N_FIELDS = 26
NUM_CATEGORIES = 1000
LATENT_DIM = 8
DIM = 64
BATCH = 16384
EPS = 1e-4

NON_CATEG = DIM - N_FIELDS
EMB_COLS = N_FIELDS * LATENT_DIM
OUT_COLS = NON_CATEG + EMB_COLS
GROUP = 16

NUM_WORKERS = 32
ROWS_PER_WORKER = BATCH // NUM_WORKERS
SUB_ROWS = 128
SUBS = ROWS_PER_WORKER // SUB_ROWS
IDX_MINOR = 128
IDX_ROWS_PER_SUB = SUB_ROWS * GROUP // IDX_MINOR
IDX_ROWS = ROWS_PER_WORKER * GROUP // IDX_MINOR
BURST = SUB_ROWS * GROUP

CONCAT_BLOCK = 256


def _prep_body(x_ref, raw_ref, tab_ref, fidxa_ref, fidxb_ref):
  tab_ref[...] = EPS + (1.0 - 2.0 * EPS) * jax.nn.sigmoid(raw_ref[...])
  x = x_ref[...]
  cols = lax.broadcasted_iota(jnp.int32, (1, GROUP), 1)
  fidxa_ref[...] = x[:, :GROUP].astype(jnp.int32) + NUM_CATEGORIES * cols
  idxb = x[:, GROUP:2 * GROUP].astype(jnp.int32) + NUM_CATEGORIES * (
      GROUP + cols)
  fidxb_ref[...] = jnp.where(cols < N_FIELDS - GROUP, idxb, 0)


def _sc_body(tab2, fidxa_hbm, fidxb_hbm, emba_hbm, embb_hbm, idxa_v, idxb_v,
             rows_a, rows_b, sem):
  wid = lax.axis_index("s") * 2 + lax.axis_index("c")
  pltpu.sync_copy(fidxa_hbm.at[wid], idxa_v)
  pltpu.sync_copy(fidxb_hbm.at[wid], idxb_v)
  for s in range(SUBS):
    base = (wid * ROWS_PER_WORKER + s * SUB_ROWS) * GROUP
    copies = []
    for idx_v, rows_v in ((idxa_v, rows_a), (idxb_v, rows_b)):
      for j in range(IDX_ROWS_PER_SUB):
        copies.append(pltpu.async_copy(
            tab2.at[idx_v.at[s * IDX_ROWS_PER_SUB + j]],
            rows_v.at[pl.ds(j * IDX_MINOR, IDX_MINOR)],
            sem,
        ))
    for c in copies:
      c.wait()
    pltpu.sync_copy(rows_a, emba_hbm.at[pl.ds(base, BURST)])
    pltpu.sync_copy(rows_b, embb_hbm.at[pl.ds(base, BURST)])


def _concat_body(x_ref, emba_ref, embb_ref, out_ref):
  out_ref[:, :NON_CATEG] = x_ref[:, N_FIELDS:]
  out_ref[:, NON_CATEG:NON_CATEG + GROUP * LATENT_DIM] = emba_ref[...]
  out_ref[:, NON_CATEG + GROUP * LATENT_DIM:] = (
      embb_ref[:, :(N_FIELDS - GROUP) * LATENT_DIM])


@jax.jit
def kernel(X, raw_emb_tables):
  raw_flat = raw_emb_tables.reshape(1625, 128)
  tab_flat, fidxa, fidxb = pl.pallas_call(
      _prep_body,
      out_shape=[
          jax.ShapeDtypeStruct((1625, 128), jnp.float32),
          jax.ShapeDtypeStruct((BATCH, GROUP), jnp.int32),
          jax.ShapeDtypeStruct((BATCH, GROUP), jnp.int32),
      ],
  )(X, raw_flat)
  tab2 = tab_flat.reshape(N_FIELDS * NUM_CATEGORIES, LATENT_DIM)
  fidxa3 = fidxa.reshape(NUM_WORKERS, IDX_ROWS, IDX_MINOR)
  fidxb3 = fidxb.reshape(NUM_WORKERS, IDX_ROWS, IDX_MINOR)

  mesh = plsc.VectorSubcoreMesh(core_axis_name="c", subcore_axis_name="s")
  emba, embb = pl.kernel(
      _sc_body,
      out_type=[
          jax.ShapeDtypeStruct((BATCH * GROUP, LATENT_DIM), jnp.float32),
          jax.ShapeDtypeStruct((BATCH * GROUP, LATENT_DIM), jnp.float32),
      ],
      mesh=mesh,
      compiler_params=pltpu.CompilerParams(use_tc_tiling_on_sc=False),
      scratch_types=[
          pltpu.VMEM((IDX_ROWS, IDX_MINOR), jnp.int32),
          pltpu.VMEM((IDX_ROWS, IDX_MINOR), jnp.int32),
          pltpu.VMEM((BURST, LATENT_DIM), jnp.float32),
          pltpu.VMEM((BURST, LATENT_DIM), jnp.float32),
          pltpu.SemaphoreType.DMA,
      ],
  )(tab2, fidxa3, fidxb3)
  emba = emba.reshape(BATCH, GROUP * LATENT_DIM)
  embb = embb.reshape(BATCH, GROUP * LATENT_DIM)

  nblk = BATCH // CONCAT_BLOCK
  return pl.pallas_call(
      _concat_body,
      grid=(nblk,),
      in_specs=[
          pl.BlockSpec((CONCAT_BLOCK, DIM), lambda k: (k, 0)),
          pl.BlockSpec((CONCAT_BLOCK, GROUP * LATENT_DIM), lambda k: (k, 0)),
          pl.BlockSpec((CONCAT_BLOCK, GROUP * LATENT_DIM), lambda k: (k, 0)),
      ],
      out_specs=pl.BlockSpec((CONCAT_BLOCK, OUT_COLS), lambda k: (k, 0)),
      out_shape=jax.ShapeDtypeStruct((BATCH, OUT_COLS), jnp.float32),
  )(X, emba, embb)


# speedup vs baseline: 12.3036x; 1.0077x over previous
import jax
import jax.numpy as jnp
from jax import lax
from jax.experimental import pallas as pl
from jax.experimental.pallas import tpu as pltpu
from jax.experimental.pallas import tpu_sc as plsc

N_FIELDS = 26
NUM_CATEGORIES = 1000
LATENT_DIM = 8
DIM = 64
BATCH = 16384
EPS = 1e-4

NON_CATEG = DIM - N_FIELDS
EMB_COLS = N_FIELDS * LATENT_DIM
OUT_COLS = NON_CATEG + EMB_COLS
GROUP = 16

NUM_WORKERS = 32
ROWS_PER_WORKER = BATCH // NUM_WORKERS
SUB_ROWS = 128
SUBS = ROWS_PER_WORKER // SUB_ROWS
IDX_MINOR = 128
IDX_ROWS_PER_SUB = SUB_ROWS * GROUP // IDX_MINOR
IDX_ROWS = ROWS_PER_WORKER * GROUP // IDX_MINOR
BURST = SUB_ROWS * GROUP

CONCAT_BLOCK = 256


def _prep_body(x_ref, raw_ref, tab_ref, fidxa_ref, fidxb_ref):
  tab_ref[...] = EPS + (1.0 - 2.0 * EPS) * jax.nn.sigmoid(raw_ref[...])
  x = x_ref[...]
  cols = lax.broadcasted_iota(jnp.int32, (1, GROUP), 1)
  fidxa_ref[...] = x[:, :GROUP].astype(jnp.int32) + NUM_CATEGORIES * cols
  idxb = x[:, GROUP:2 * GROUP].astype(jnp.int32) + NUM_CATEGORIES * (
      GROUP + cols)
  fidxb_ref[...] = jnp.where(cols < N_FIELDS - GROUP, idxb, 0)


NBUF = 3


def _sc_body(tab2, fidxa_hbm, fidxb_hbm, emba_hbm, embb_hbm, idxa_v, idxb_v,
             *bufs_and_sems):
  rows_a = bufs_and_sems[:NBUF]
  rows_b = bufs_and_sems[NBUF:2 * NBUF]
  sg = bufs_and_sems[2 * NBUF:3 * NBUF]
  sw = bufs_and_sems[3 * NBUF:4 * NBUF]
  wid = lax.axis_index("s") * 2 + lax.axis_index("c")
  pltpu.sync_copy(fidxa_hbm.at[wid], idxa_v)
  pltpu.sync_copy(fidxb_hbm.at[wid], idxb_v)

  def fire_gather(s):
    p = s % NBUF
    sl = pl.ds(s * BURST, BURST)
    return (
        pltpu.async_copy(tab2.at[idxa_v.at[sl]], rows_a[p], sg[p]),
        pltpu.async_copy(tab2.at[idxb_v.at[sl]], rows_b[p], sg[p]),
    )

  def fire_write(s):
    p = s % NBUF
    base = (wid * ROWS_PER_WORKER + s * SUB_ROWS) * GROUP
    return (
        pltpu.async_copy(rows_a[p], emba_hbm.at[pl.ds(base, BURST)], sw[p]),
        pltpu.async_copy(rows_b[p], embb_hbm.at[pl.ds(base, BURST)], sw[p]),
    )

  gathers = {s: fire_gather(s) for s in range(min(NBUF, SUBS))}
  writes = {}
  for s in range(SUBS):
    for c in gathers[s]:
      c.wait()
    writes[s] = fire_write(s)
    if s + NBUF < SUBS:
      for c in writes[s]:
        c.wait()
      writes.pop(s)
      gathers[s + NBUF] = fire_gather(s + NBUF)
  for w in writes.values():
    for c in w:
      c.wait()


def _concat_body(x_ref, emba_ref, embb_ref, out_ref):
  out_ref[:, :NON_CATEG] = x_ref[:, N_FIELDS:]
  out_ref[:, NON_CATEG:NON_CATEG + GROUP * LATENT_DIM] = emba_ref[...]
  out_ref[:, NON_CATEG + GROUP * LATENT_DIM:] = (
      embb_ref[:, :(N_FIELDS - GROUP) * LATENT_DIM])


@jax.jit
def kernel(X, raw_emb_tables):
  raw_flat = raw_emb_tables.reshape(1625, 128)
  tab_flat, fidxa, fidxb = pl.pallas_call(
      _prep_body,
      out_shape=[
          jax.ShapeDtypeStruct((1625, 128), jnp.float32),
          jax.ShapeDtypeStruct((BATCH, GROUP), jnp.int32),
          jax.ShapeDtypeStruct((BATCH, GROUP), jnp.int32),
      ],
  )(X, raw_flat)
  tab2 = tab_flat.reshape(N_FIELDS * NUM_CATEGORIES, LATENT_DIM)
  fidxa3 = fidxa.reshape(NUM_WORKERS, ROWS_PER_WORKER * GROUP)
  fidxb3 = fidxb.reshape(NUM_WORKERS, ROWS_PER_WORKER * GROUP)

  mesh = plsc.VectorSubcoreMesh(core_axis_name="c", subcore_axis_name="s")
  emba, embb = pl.kernel(
      _sc_body,
      out_type=[
          jax.ShapeDtypeStruct((BATCH * GROUP, LATENT_DIM), jnp.float32),
          jax.ShapeDtypeStruct((BATCH * GROUP, LATENT_DIM), jnp.float32),
      ],
      mesh=mesh,
      compiler_params=pltpu.CompilerParams(use_tc_tiling_on_sc=False),
      scratch_types=(
          [
              pltpu.VMEM((ROWS_PER_WORKER * GROUP,), jnp.int32),
              pltpu.VMEM((ROWS_PER_WORKER * GROUP,), jnp.int32),
          ]
          + [pltpu.VMEM((BURST, LATENT_DIM), jnp.float32)] * (2 * NBUF)
          + [pltpu.SemaphoreType.DMA] * (2 * NBUF)
      ),
  )(tab2, fidxa3, fidxb3)
  emba = emba.reshape(BATCH, GROUP * LATENT_DIM)
  embb = embb.reshape(BATCH, GROUP * LATENT_DIM)

  nblk = BATCH // CONCAT_BLOCK
  return pl.pallas_call(
      _concat_body,
      grid=(nblk,),
      in_specs=[
          pl.BlockSpec((CONCAT_BLOCK, DIM), lambda k: (k, 0)),
          pl.BlockSpec((CONCAT_BLOCK, GROUP * LATENT_DIM), lambda k: (k, 0)),
          pl.BlockSpec((CONCAT_BLOCK, GROUP * LATENT_DIM), lambda k: (k, 0)),
      ],
      out_specs=pl.BlockSpec((CONCAT_BLOCK, OUT_COLS), lambda k: (k, 0)),
      out_shape=jax.ShapeDtypeStruct((BATCH, OUT_COLS), jnp.float32),
  )(X, emba, embb)


# speedup vs baseline: 39.1395x; 3.1811x over previous
import jax
import jax.numpy as jnp
from jax import lax
from jax.experimental import pallas as pl
from jax.experimental.pallas import tpu as pltpu
from jax.experimental.pallas import tpu_sc as plsc

N_FIELDS = 26
NUM_CATEGORIES = 1000
LATENT_DIM = 8
DIM = 64
BATCH = 16384
EPS = 1e-4

NON_CATEG = DIM - N_FIELDS
EMB_COLS = N_FIELDS * LATENT_DIM
OUT_COLS = NON_CATEG + EMB_COLS
GROUP = 16

NUM_WORKERS = 32
QUARTERS = 4
TASK_ROWS = BATCH // QUARTERS
LANES = 16
TASK_GROUPS = TASK_ROWS // LANES

PREP_BLOCK = 1024
CONCAT_BLOCK = 256


def _prep_body(x_ref, raw_ref, tab_ref, idxt_ref):
  @pl.when(pl.program_id(0) == 0)
  def _():
    tab_ref[...] = EPS + (1.0 - 2.0 * EPS) * jax.nn.sigmoid(raw_ref[...])

  idxt_ref[...] = x_ref[:, :N_FIELDS].astype(jnp.int32).T


def _sc_body(tab2, idxt_hbm, emba_hbm, embb_hbm, tab_v, idx_v, obuf0, obuf1,
             sw0, sw1):
  wid = lax.axis_index("s") * 2 + lax.axis_index("c")
  obufs = (obuf0, obuf1)
  sems = (sw0, sw1)
  iota = lax.broadcasted_iota(jnp.int32, (LANES,), 0)
  lsplats = [jnp.full((LANES,), l, jnp.int32) for l in range(LATENT_DIM)]
  pending = [None, None]

  def run_task(p, f, q, emb_hbm):
    pltpu.sync_copy(tab2.at[pl.ds(pl.multiple_of(f * NUM_CATEGORIES, 8),
                                  NUM_CATEGORIES)], tab_v)
    pltpu.sync_copy(idxt_hbm.at[f, pl.ds(pl.multiple_of(q * TASK_ROWS, 8),
                                         TASK_ROWS)], idx_v)
    if pending[p] is not None:
      pending[p].wait()
    obuf = obufs[p]

    def group(g, _):
      iv = idx_v[pl.ds(g * LANES, LANES)]
      rows = g * LANES + iota
      for l in range(LATENT_DIM):
        v = plsc.load_gather(tab_v, [iv, lsplats[l]])
        plsc.store_scatter(obuf, [rows, lsplats[l]], v)
      return _

    lax.fori_loop(0, TASK_GROUPS, group, None)
    col = pl.multiple_of((f % GROUP) * LATENT_DIM, 8)
    row0 = pl.multiple_of(q * TASK_ROWS, 8)
    pending[p] = pltpu.async_copy(
        obuf,
        emb_hbm.at[pl.ds(row0, TASK_ROWS), pl.ds(col, LATENT_DIM)],
        sems[p],
    )

  for i in range(2):
    a = 2 * wid + i
    run_task(i % 2, a % GROUP, a // GROUP, emba_hbm)
  b = wid
  run_task(0, GROUP + b % 10, b // 10, embb_hbm)
  pending[1].wait()
  pending[1] = None

  @pl.when(wid < 8)
  def _():
    b2 = 32 + wid
    run_task(1, GROUP + b2 % 10, b2 // 10, embb_hbm)
    pending[1].wait()
    pending[1] = None

  pending[0].wait()


def _concat_body(x_ref, emba_ref, embb_ref, out_ref):
  out_ref[:, :NON_CATEG] = x_ref[:, N_FIELDS:]
  out_ref[:, NON_CATEG:NON_CATEG + GROUP * LATENT_DIM] = emba_ref[...]
  out_ref[:, NON_CATEG + GROUP * LATENT_DIM:] = (
      embb_ref[:, :(N_FIELDS - GROUP) * LATENT_DIM])


@jax.jit
def kernel(X, raw_emb_tables):
  raw_flat = raw_emb_tables.reshape(1625, 128)
  nprep = BATCH // PREP_BLOCK
  tab_flat, idxt = pl.pallas_call(
      _prep_body,
      grid=(nprep,),
      in_specs=[
          pl.BlockSpec((PREP_BLOCK, DIM), lambda k: (k, 0)),
          pl.BlockSpec((1625, 128), lambda k: (0, 0)),
      ],
      out_specs=[
          pl.BlockSpec((1625, 128), lambda k: (0, 0)),
          pl.BlockSpec((N_FIELDS, PREP_BLOCK), lambda k: (0, k)),
      ],
      out_shape=[
          jax.ShapeDtypeStruct((1625, 128), jnp.float32),
          jax.ShapeDtypeStruct((N_FIELDS, BATCH), jnp.int32),
      ],
  )(X, raw_flat)
  tab2 = tab_flat.reshape(N_FIELDS * NUM_CATEGORIES, LATENT_DIM)

  mesh = plsc.VectorSubcoreMesh(core_axis_name="c", subcore_axis_name="s")
  emba, embb = pl.kernel(
      _sc_body,
      out_type=[
          jax.ShapeDtypeStruct((BATCH, GROUP * LATENT_DIM), jnp.float32),
          jax.ShapeDtypeStruct((BATCH, GROUP * LATENT_DIM), jnp.float32),
      ],
      mesh=mesh,
      compiler_params=pltpu.CompilerParams(
          use_tc_tiling_on_sc=False, needs_layout_passes=False),
      scratch_types=[
          pltpu.VMEM((NUM_CATEGORIES, LATENT_DIM), jnp.float32),
          pltpu.VMEM((TASK_ROWS,), jnp.int32),
          pltpu.VMEM((TASK_ROWS, LATENT_DIM), jnp.float32),
          pltpu.VMEM((TASK_ROWS, LATENT_DIM), jnp.float32),
          pltpu.SemaphoreType.DMA,
          pltpu.SemaphoreType.DMA,
      ],
  )(tab2, idxt)

  nblk = BATCH // CONCAT_BLOCK
  return pl.pallas_call(
      _concat_body,
      grid=(nblk,),
      in_specs=[
          pl.BlockSpec((CONCAT_BLOCK, DIM), lambda k: (k, 0)),
          pl.BlockSpec((CONCAT_BLOCK, GROUP * LATENT_DIM), lambda k: (k, 0)),
          pl.BlockSpec((CONCAT_BLOCK, GROUP * LATENT_DIM), lambda k: (k, 0)),
      ],
      out_specs=pl.BlockSpec((CONCAT_BLOCK, OUT_COLS), lambda k: (k, 0)),
      out_shape=jax.ShapeDtypeStruct((BATCH, OUT_COLS), jnp.float32),
  )(X, emba, embb)


# speedup vs baseline: 52.6317x; 1.3447x over previous
import jax
import jax.numpy as jnp
from jax import lax
from jax.experimental import pallas as pl
from jax.experimental.pallas import tpu as pltpu
from jax.experimental.pallas import tpu_sc as plsc

N_FIELDS = 26
NUM_CATEGORIES = 1000
LATENT_DIM = 8
DIM = 64
BATCH = 16384
EPS = 1e-4

NON_CATEG = DIM - N_FIELDS
OUT_COLS = NON_CATEG + N_FIELDS * LATENT_DIM
GROUP = 16

QUARTERS = 4
TASK_ROWS = BATCH // QUARTERS
LANES = 16
TASK_GROUPS = TASK_ROWS // LANES

PREP_BLOCK = 2048
CONCAT_BLOCK = 2048


def _prep_body(xt_ref, raw_ref, tab_ref, idxt_ref):
  @pl.when(pl.program_id(0) == 0)
  def _():
    tab_ref[...] = EPS + (1.0 - 2.0 * EPS) * jax.nn.sigmoid(raw_ref[...])

  idxt_ref[...] = xt_ref[:N_FIELDS, :].astype(jnp.int32)


def _sc_body(tab2, idxt_hbm, emba_hbm, embb_hbm, tab_v, idx_v, obuf0, obuf1,
             sw0, sw1):
  wid = lax.axis_index("s") * 2 + lax.axis_index("c")
  obufs = (obuf0, obuf1)
  sems = (sw0, sw1)
  lsplats = [jnp.full((LANES,), l, jnp.int32) for l in range(LATENT_DIM)]
  pending = [None, None]

  def run_task(p, f, q, emb_hbm):
    pltpu.sync_copy(tab2.at[pl.ds(pl.multiple_of(f * NUM_CATEGORIES, 8),
                                  NUM_CATEGORIES)], tab_v)
    pltpu.sync_copy(idxt_hbm.at[f, pl.ds(pl.multiple_of(q * TASK_ROWS, 8),
                                         TASK_ROWS)], idx_v)
    if pending[p] is not None:
      pending[p].wait()
    obuf = obufs[p]

    def group(g, _):
      iv = idx_v[pl.ds(g * LANES, LANES)]
      for l in range(LATENT_DIM):
        v = plsc.load_gather(tab_v, [iv, lsplats[l]])
        obuf[l, pl.ds(g * LANES, LANES)] = v
      return _

    lax.fori_loop(0, TASK_GROUPS, group, None)
    row0 = pl.multiple_of((f % GROUP) * LATENT_DIM, 8)
    col0 = pl.multiple_of(q * TASK_ROWS, 8)
    pending[p] = pltpu.async_copy(
        obuf,
        emb_hbm.at[pl.ds(row0, LATENT_DIM), pl.ds(col0, TASK_ROWS)],
        sems[p],
    )

  for i in range(2):
    a = 2 * wid + i
    run_task(i % 2, a % GROUP, a // GROUP, emba_hbm)
  b = wid
  run_task(0, GROUP + b % 10, b // 10, embb_hbm)
  pending[1].wait()
  pending[1] = None

  @pl.when(wid < 8)
  def _():
    b2 = 32 + wid
    run_task(1, GROUP + b2 % 10, b2 // 10, embb_hbm)
    pending[1].wait()
    pending[1] = None

  pending[0].wait()


def _concat_body(xt_ref, emba_ref, embb_ref, out_ref):
  out_ref[:NON_CATEG, :] = xt_ref[N_FIELDS:, :]
  out_ref[NON_CATEG:NON_CATEG + GROUP * LATENT_DIM, :] = emba_ref[...]
  out_ref[NON_CATEG + GROUP * LATENT_DIM:, :] = (
      embb_ref[:(N_FIELDS - GROUP) * LATENT_DIM, :])


@jax.jit
def kernel(X, raw_emb_tables):
  XT = X.T
  raw_flat = raw_emb_tables.reshape(1625, 128)
  nprep = BATCH // PREP_BLOCK
  tab_flat, idxt = pl.pallas_call(
      _prep_body,
      grid=(nprep,),
      in_specs=[
          pl.BlockSpec((DIM, PREP_BLOCK), lambda k: (0, k)),
          pl.BlockSpec((1625, 128), lambda k: (0, 0)),
      ],
      out_specs=[
          pl.BlockSpec((1625, 128), lambda k: (0, 0)),
          pl.BlockSpec((N_FIELDS, PREP_BLOCK), lambda k: (0, k)),
      ],
      out_shape=[
          jax.ShapeDtypeStruct((1625, 128), jnp.float32),
          jax.ShapeDtypeStruct((N_FIELDS, BATCH), jnp.int32),
      ],
  )(XT, raw_flat)
  tab2 = tab_flat.reshape(N_FIELDS * NUM_CATEGORIES, LATENT_DIM)

  mesh = plsc.VectorSubcoreMesh(core_axis_name="c", subcore_axis_name="s")
  embat, embbt = pl.kernel(
      _sc_body,
      out_type=[
          jax.ShapeDtypeStruct((GROUP * LATENT_DIM, BATCH), jnp.float32),
          jax.ShapeDtypeStruct((GROUP * LATENT_DIM, BATCH), jnp.float32),
      ],
      mesh=mesh,
      compiler_params=pltpu.CompilerParams(
          use_tc_tiling_on_sc=False, needs_layout_passes=False),
      scratch_types=[
          pltpu.VMEM((NUM_CATEGORIES, LATENT_DIM), jnp.float32),
          pltpu.VMEM((TASK_ROWS,), jnp.int32),
          pltpu.VMEM((LATENT_DIM, TASK_ROWS), jnp.float32),
          pltpu.VMEM((LATENT_DIM, TASK_ROWS), jnp.float32),
          pltpu.SemaphoreType.DMA,
          pltpu.SemaphoreType.DMA,
      ],
  )(tab2, idxt)

  nblk = BATCH // CONCAT_BLOCK
  outt = pl.pallas_call(
      _concat_body,
      grid=(nblk,),
      in_specs=[
          pl.BlockSpec((DIM, CONCAT_BLOCK), lambda k: (0, k)),
          pl.BlockSpec((GROUP * LATENT_DIM, CONCAT_BLOCK), lambda k: (0, k)),
          pl.BlockSpec((GROUP * LATENT_DIM, CONCAT_BLOCK), lambda k: (0, k)),
      ],
      out_specs=pl.BlockSpec((OUT_COLS, CONCAT_BLOCK), lambda k: (0, k)),
      out_shape=jax.ShapeDtypeStruct((OUT_COLS, BATCH), jnp.float32),
  )(XT, embat, embbt)
  return outt.T


# speedup vs baseline: 79.1770x; 1.5044x over previous
import jax
import jax.numpy as jnp
from jax import lax
from jax.experimental import pallas as pl
from jax.experimental.pallas import tpu as pltpu
from jax.experimental.pallas import tpu_sc as plsc

N_FIELDS = 26
NUM_CATEGORIES = 1000
LATENT_DIM = 8
DIM = 64
BATCH = 16384
EPS = 1e-4

NON_CATEG = DIM - N_FIELDS
OUT_COLS = NON_CATEG + N_FIELDS * LATENT_DIM
GROUP = 16

QUARTERS = 8
TASK_ROWS = BATCH // QUARTERS
LANES = 16
TASK_GROUPS = TASK_ROWS // LANES

PREP_BLOCK = 2048
CONCAT_BLOCK = 2048


def _prep_body(xt_ref, raw_ref, tab_ref, idxt_ref):
  @pl.when(pl.program_id(0) == 0)
  def _():
    tab_ref[...] = EPS + (1.0 - 2.0 * EPS) * jax.nn.sigmoid(raw_ref[...])

  idxt_ref[...] = xt_ref[:N_FIELDS, :].astype(jnp.int32)


def _sc_body(tab2, idxt_hbm, emba_hbm, embb_hbm, tab_v0, tab_v1, idx_v0,
             idx_v1, obuf0, obuf1, sg0, sg1, sw0, sw1):
  wid = lax.axis_index("s") * 2 + lax.axis_index("c")
  tab_vs = (tab_v0, tab_v1)
  idx_vs = (idx_v0, idx_v1)
  obufs = (obuf0, obuf1)
  sg = (sg0, sg1)
  sw = (sw0, sw1)
  lsplats = [jnp.full((LANES,), l, jnp.int32) for l in range(LATENT_DIM)]
  pending = [None, None]

  defs = []
  for i in range(4):
    a = 4 * wid + i
    defs.append((a % GROUP, a // GROUP, emba_hbm))
  for i in range(2):
    b = 2 * wid + i
    defs.append((GROUP + b % 10, b // 10, embb_hbm))
  b6 = 64 + wid
  defs.append((GROUP + b6 % 10, jnp.minimum(b6 // 10, QUARTERS - 1),
               embb_hbm))

  def fire_stage(k):
    p = k % 2
    f, q, _ = defs[k]
    c1 = pltpu.async_copy(
        tab2.at[pl.ds(pl.multiple_of(f * NUM_CATEGORIES, 8),
                      NUM_CATEGORIES)], tab_vs[p], sg[p])
    c2 = pltpu.async_copy(
        idxt_hbm.at[f, pl.ds(pl.multiple_of(q * TASK_ROWS, 8), TASK_ROWS)],
        idx_vs[p], sg[p])
    return (c1, c2)

  def compute(k):
    p = k % 2
    tab_v, idx_v, obuf = tab_vs[p], idx_vs[p], obufs[p]

    @plsc.parallel_loop(0, TASK_GROUPS, unroll=2)
    def _(g):
      iv = idx_v[pl.ds(g * LANES, LANES)]
      vs = [plsc.load_gather(tab_v, [iv, lsplats[l]])
            for l in range(LATENT_DIM)]
      for l in range(LATENT_DIM):
        obuf[l, pl.ds(g * LANES, LANES)] = vs[l]

  def fire_write(k):
    p = k % 2
    f, q, emb_hbm = defs[k]
    row0 = pl.multiple_of((f % GROUP) * LATENT_DIM, 8)
    col0 = pl.multiple_of(q * TASK_ROWS, 8)
    return pltpu.async_copy(
        obufs[p],
        emb_hbm.at[pl.ds(row0, LATENT_DIM), pl.ds(col0, TASK_ROWS)],
        sw[p],
    )

  stage_pending = {0: fire_stage(0)}
  for k in range(7):
    if k + 1 < 7:
      stage_pending[k + 1] = fire_stage(k + 1)
    for c in stage_pending.pop(k):
      c.wait()
    if k == 6:
      break
    p = k % 2
    if pending[p] is not None:
      pending[p].wait()
    compute(k)
    pending[p] = fire_write(k)

  pending[0].wait()
  pending[0] = None

  @pl.when(wid < 16)
  def _():
    compute(6)
    fire_write(6).wait()

  pending[1].wait()


def _concat_body(xt_ref, emba_ref, embb_ref, out_ref):
  out_ref[:NON_CATEG, :] = xt_ref[N_FIELDS:, :]
  out_ref[NON_CATEG:NON_CATEG + GROUP * LATENT_DIM, :] = emba_ref[...]
  out_ref[NON_CATEG + GROUP * LATENT_DIM:, :] = (
      embb_ref[:(N_FIELDS - GROUP) * LATENT_DIM, :])


@jax.jit
def kernel(X, raw_emb_tables):
  XT = X.T
  raw_flat = raw_emb_tables.reshape(1625, 128)
  nprep = BATCH // PREP_BLOCK
  tab_flat, idxt = pl.pallas_call(
      _prep_body,
      grid=(nprep,),
      in_specs=[
          pl.BlockSpec((DIM, PREP_BLOCK), lambda k: (0, k)),
          pl.BlockSpec((1625, 128), lambda k: (0, 0)),
      ],
      out_specs=[
          pl.BlockSpec((1625, 128), lambda k: (0, 0)),
          pl.BlockSpec((N_FIELDS, PREP_BLOCK), lambda k: (0, k)),
      ],
      out_shape=[
          jax.ShapeDtypeStruct((1625, 128), jnp.float32),
          jax.ShapeDtypeStruct((N_FIELDS, BATCH), jnp.int32),
      ],
  )(XT, raw_flat)
  tab2 = tab_flat.reshape(N_FIELDS * NUM_CATEGORIES, LATENT_DIM)

  mesh = plsc.VectorSubcoreMesh(core_axis_name="c", subcore_axis_name="s")
  embat, embbt = pl.kernel(
      _sc_body,
      out_type=[
          jax.ShapeDtypeStruct((GROUP * LATENT_DIM, BATCH), jnp.float32),
          jax.ShapeDtypeStruct((GROUP * LATENT_DIM, BATCH), jnp.float32),
      ],
      mesh=mesh,
      compiler_params=pltpu.CompilerParams(
          use_tc_tiling_on_sc=False, needs_layout_passes=False),
      scratch_types=[
          pltpu.VMEM((NUM_CATEGORIES, LATENT_DIM), jnp.float32),
          pltpu.VMEM((NUM_CATEGORIES, LATENT_DIM), jnp.float32),
          pltpu.VMEM((TASK_ROWS,), jnp.int32),
          pltpu.VMEM((TASK_ROWS,), jnp.int32),
          pltpu.VMEM((LATENT_DIM, TASK_ROWS), jnp.float32),
          pltpu.VMEM((LATENT_DIM, TASK_ROWS), jnp.float32),
          pltpu.SemaphoreType.DMA,
          pltpu.SemaphoreType.DMA,
          pltpu.SemaphoreType.DMA,
          pltpu.SemaphoreType.DMA,
      ],
  )(tab2, idxt)

  nblk = BATCH // CONCAT_BLOCK
  outt = pl.pallas_call(
      _concat_body,
      grid=(nblk,),
      in_specs=[
          pl.BlockSpec((DIM, CONCAT_BLOCK), lambda k: (0, k)),
          pl.BlockSpec((GROUP * LATENT_DIM, CONCAT_BLOCK), lambda k: (0, k)),
          pl.BlockSpec((GROUP * LATENT_DIM, CONCAT_BLOCK), lambda k: (0, k)),
      ],
      out_specs=pl.BlockSpec((OUT_COLS, CONCAT_BLOCK), lambda k: (0, k)),
      out_shape=jax.ShapeDtypeStruct((OUT_COLS, BATCH), jnp.float32),
  )(XT, embat, embbt)
  return outt.T


# speedup vs baseline: 98.1946x; 1.2402x over previous
import jax
import jax.numpy as jnp
from jax import lax
from jax.experimental import pallas as pl
from jax.experimental.pallas import tpu as pltpu
from jax.experimental.pallas import tpu_sc as plsc

N_FIELDS = 26
NUM_CATEGORIES = 1000
LATENT_DIM = 8
DIM = 64
BATCH = 16384
EPS = 1e-4

NON_CATEG = DIM - N_FIELDS
OUT_COLS = NON_CATEG + N_FIELDS * LATENT_DIM
GROUP = 16

QUARTERS = 8
TASK_ROWS = BATCH // QUARTERS
LANES = 16
TASK_GROUPS = TASK_ROWS // LANES

PREP_BLOCK = 2048
CONCAT_BLOCK = 2048


KPAD = 1024


def _prep_body(xt_ref, raw_ref, tab_ref, idxt_ref):
  @pl.when(pl.program_id(0) == 0)
  def _():
    t = EPS + (1.0 - 2.0 * EPS) * jax.nn.sigmoid(raw_ref[...])
    tp = jnp.concatenate(
        [t, jnp.zeros((N_FIELDS, LATENT_DIM, KPAD - NUM_CATEGORIES),
                      jnp.float32)], axis=-1)
    tab_ref[...] = tp.reshape(N_FIELDS, LATENT_DIM, KPAD // 128, 128)

  idxt_ref[...] = xt_ref[:N_FIELDS, :].astype(jnp.int32)


def _sc_body(tab2, idxt_hbm, emba_hbm, embb_hbm, tab_v0, tab_v1, idx_v0,
             idx_v1, obuf0, obuf1, sg0, sg1, sw0, sw1):
  wid = lax.axis_index("s") * 2 + lax.axis_index("c")
  tab_vs = (tab_v0, tab_v1)
  idx_vs = (idx_v0, idx_v1)
  obufs = (obuf0, obuf1)
  sg = (sg0, sg1)
  sw = (sw0, sw1)
  lsplats = [jnp.full((LANES,), l, jnp.int32) for l in range(LATENT_DIM)]
  pending = [None, None]

  defs = []
  for i in range(4):
    a = 4 * wid + i
    defs.append((a % GROUP, a // GROUP, emba_hbm))
  for i in range(2):
    b = 2 * wid + i
    defs.append((GROUP + b % 10, b // 10, embb_hbm))
  b6 = 64 + wid
  defs.append((GROUP + b6 % 10, jnp.minimum(b6 // 10, QUARTERS - 1),
               embb_hbm))

  def fire_stage(k):
    p = k % 2
    f, q, _ = defs[k]
    c1 = pltpu.async_copy(tab2.at[f], tab_vs[p], sg[p])
    c2 = pltpu.async_copy(
        idxt_hbm.at[f, pl.ds(pl.multiple_of(q * TASK_ROWS, 8), TASK_ROWS)],
        idx_vs[p], sg[p])
    return (c1, c2)

  def compute(k):
    p = k % 2
    tab_v, idx_v, obuf = tab_vs[p], idx_vs[p], obufs[p]

    @plsc.parallel_loop(0, TASK_GROUPS, unroll=2)
    def _(g):
      iv = idx_v[pl.ds(g * LANES, LANES)]
      hi = lax.shift_right_logical(iv, 7)
      lo = lax.bitwise_and(iv, 127)
      vs = [plsc.load_gather(tab_v, [lsplats[l], hi, lo])
            for l in range(LATENT_DIM)]
      for l in range(LATENT_DIM):
        obuf[l, pl.ds(g * LANES, LANES)] = vs[l]

  def fire_write(k):
    p = k % 2
    f, q, emb_hbm = defs[k]
    row0 = pl.multiple_of((f % GROUP) * LATENT_DIM, 8)
    col0 = pl.multiple_of(q * TASK_ROWS, 8)
    return pltpu.async_copy(
        obufs[p],
        emb_hbm.at[pl.ds(row0, LATENT_DIM), pl.ds(col0, TASK_ROWS)],
        sw[p],
    )

  stage_pending = {0: fire_stage(0)}
  for k in range(7):
    if k + 1 < 7:
      stage_pending[k + 1] = fire_stage(k + 1)
    for c in stage_pending.pop(k):
      c.wait()
    if k == 6:
      break
    p = k % 2
    if pending[p] is not None:
      pending[p].wait()
    compute(k)
    pending[p] = fire_write(k)

  pending[0].wait()
  pending[0] = None

  @pl.when(wid < 16)
  def _():
    compute(6)
    fire_write(6).wait()

  pending[1].wait()


def _concat_body(xt_ref, emba_ref, embb_ref, out_ref):
  out_ref[:NON_CATEG, :] = xt_ref[N_FIELDS:, :]
  out_ref[NON_CATEG:NON_CATEG + GROUP * LATENT_DIM, :] = emba_ref[...]
  out_ref[NON_CATEG + GROUP * LATENT_DIM:, :] = (
      embb_ref[:(N_FIELDS - GROUP) * LATENT_DIM, :])


@jax.jit
def kernel(X, raw_emb_tables):
  XT = X.T
  raw_t = raw_emb_tables.transpose(0, 2, 1)
  nprep = BATCH // PREP_BLOCK
  tab2, idxt = pl.pallas_call(
      _prep_body,
      grid=(nprep,),
      in_specs=[
          pl.BlockSpec((DIM, PREP_BLOCK), lambda k: (0, k)),
          pl.BlockSpec((N_FIELDS, LATENT_DIM, NUM_CATEGORIES),
                       lambda k: (0, 0, 0)),
      ],
      out_specs=[
          pl.BlockSpec((N_FIELDS, LATENT_DIM, KPAD // 128, 128),
                       lambda k: (0, 0, 0, 0)),
          pl.BlockSpec((N_FIELDS, PREP_BLOCK), lambda k: (0, k)),
      ],
      out_shape=[
          jax.ShapeDtypeStruct((N_FIELDS, LATENT_DIM, KPAD // 128, 128),
                               jnp.float32),
          jax.ShapeDtypeStruct((N_FIELDS, BATCH), jnp.int32),
      ],
  )(XT, raw_t)

  mesh = plsc.VectorSubcoreMesh(core_axis_name="c", subcore_axis_name="s")
  embat, embbt = pl.kernel(
      _sc_body,
      out_type=[
          jax.ShapeDtypeStruct((GROUP * LATENT_DIM, BATCH), jnp.float32),
          jax.ShapeDtypeStruct((GROUP * LATENT_DIM, BATCH), jnp.float32),
      ],
      mesh=mesh,
      compiler_params=pltpu.CompilerParams(
          use_tc_tiling_on_sc=False, needs_layout_passes=False),
      scratch_types=[
          pltpu.VMEM((LATENT_DIM, KPAD // 128, 128), jnp.float32),
          pltpu.VMEM((LATENT_DIM, KPAD // 128, 128), jnp.float32),
          pltpu.VMEM((TASK_ROWS,), jnp.int32),
          pltpu.VMEM((TASK_ROWS,), jnp.int32),
          pltpu.VMEM((LATENT_DIM, TASK_ROWS), jnp.float32),
          pltpu.VMEM((LATENT_DIM, TASK_ROWS), jnp.float32),
          pltpu.SemaphoreType.DMA,
          pltpu.SemaphoreType.DMA,
          pltpu.SemaphoreType.DMA,
          pltpu.SemaphoreType.DMA,
      ],
  )(tab2, idxt)

  nblk = BATCH // CONCAT_BLOCK
  outt = pl.pallas_call(
      _concat_body,
      grid=(nblk,),
      in_specs=[
          pl.BlockSpec((DIM, CONCAT_BLOCK), lambda k: (0, k)),
          pl.BlockSpec((GROUP * LATENT_DIM, CONCAT_BLOCK), lambda k: (0, k)),
          pl.BlockSpec((GROUP * LATENT_DIM, CONCAT_BLOCK), lambda k: (0, k)),
      ],
      out_specs=pl.BlockSpec((OUT_COLS, CONCAT_BLOCK), lambda k: (0, k)),
      out_shape=jax.ShapeDtypeStruct((OUT_COLS, BATCH), jnp.float32),
  )(XT, embat, embbt)
  return outt.T


# speedup vs baseline: 136.9405x; 1.3946x over previous
import jax
import jax.numpy as jnp
from jax import lax
from jax.experimental import pallas as pl
from jax.experimental.pallas import tpu as pltpu
from jax.experimental.pallas import tpu_sc as plsc

N_FIELDS = 26
NUM_CATEGORIES = 1000
LATENT_DIM = 8
DIM = 64
BATCH = 16384
EPS = 1e-4

NON_CATEG = DIM - N_FIELDS
OUT_COLS = NON_CATEG + N_FIELDS * LATENT_DIM
GROUP = 16

QUARTERS = 8
TASK_ROWS = BATCH // QUARTERS
LANES = 16
TASK_GROUPS = TASK_ROWS // LANES

PREP_BLOCK = 2048
CONCAT_BLOCK = 2048


KPAD = 1024


def _prep_body(xt_ref, raw_ref, tab_ref, idxt_ref):
  @pl.when(pl.program_id(0) == 0)
  def _():
    t = EPS + (1.0 - 2.0 * EPS) * jax.nn.sigmoid(raw_ref[...])
    tp = jnp.concatenate(
        [t, jnp.zeros((N_FIELDS, LATENT_DIM, KPAD - NUM_CATEGORIES),
                      jnp.float32)], axis=-1)
    tab_ref[...] = tp.reshape(N_FIELDS, LATENT_DIM, KPAD // 128, 128)

  idxt_ref[...] = xt_ref[:N_FIELDS, :].astype(jnp.int32).reshape(
      N_FIELDS, PREP_BLOCK // 128, 128)


def _sc_body(tab2, idxt_hbm, emba_hbm, embb_hbm, tab_v0, tab_v1, idx_v0,
             idx_v1, obuf0, obuf1, sg0, sg1, sw0, sw1):
  wid = lax.axis_index("s") * 2 + lax.axis_index("c")
  tab_vs = (tab_v0, tab_v1)
  idx_vs = (idx_v0, idx_v1)
  obufs = (obuf0, obuf1)
  sg = (sg0, sg1)
  sw = (sw0, sw1)
  lsplats = [jnp.full((LANES,), l, jnp.int32) for l in range(LATENT_DIM)]
  pending = [None, None]

  defs = []
  for i in range(4):
    a = 4 * wid + i
    defs.append((a % GROUP, a // GROUP, emba_hbm))
  for i in range(2):
    b = 2 * wid + i
    defs.append((GROUP + b % 10, b // 10, embb_hbm))
  b6 = 64 + wid
  defs.append((GROUP + b6 % 10, jnp.minimum(b6 // 10, QUARTERS - 1),
               embb_hbm))

  def fire_stage(k):
    p = k % 2
    f, q, _ = defs[k]
    c1 = pltpu.async_copy(tab2.at[f], tab_vs[p], sg[p])
    c2 = pltpu.async_copy(
        idxt_hbm.at[f, pl.ds(pl.multiple_of(q * (TASK_ROWS // 128), 8),
                             TASK_ROWS // 128)],
        idx_vs[p], sg[p])
    return (c1, c2)

  def compute(k):
    p = k % 2
    tab_v, idx_v, obuf = tab_vs[p], idx_vs[p], obufs[p]

    @plsc.parallel_loop(0, TASK_GROUPS, unroll=2)
    def _(g):
      gr = g // 8
      gc = (g % 8) * LANES
      iv = idx_v[gr, pl.ds(gc, LANES)]
      hi = lax.shift_right_logical(iv, 7)
      lo = lax.bitwise_and(iv, 127)
      vs = [plsc.load_gather(tab_v, [lsplats[l], hi, lo])
            for l in range(LATENT_DIM)]
      for l in range(LATENT_DIM):
        obuf[l, gr, pl.ds(gc, LANES)] = vs[l]

  def fire_write(k):
    p = k % 2
    f, q, emb_hbm = defs[k]
    row0 = pl.multiple_of((f % GROUP) * LATENT_DIM, 8)
    col0 = pl.multiple_of(q * (TASK_ROWS // 128), 8)
    return pltpu.async_copy(
        obufs[p],
        emb_hbm.at[pl.ds(row0, LATENT_DIM), pl.ds(col0, TASK_ROWS // 128)],
        sw[p],
    )

  stage_pending = {0: fire_stage(0)}
  for k in range(7):
    if k + 1 < 7:
      stage_pending[k + 1] = fire_stage(k + 1)
    for c in stage_pending.pop(k):
      c.wait()
    if k == 6:
      break
    p = k % 2
    if pending[p] is not None:
      pending[p].wait()
    compute(k)
    pending[p] = fire_write(k)

  pending[0].wait()
  pending[0] = None

  @pl.when(wid < 16)
  def _():
    compute(6)
    fire_write(6).wait()

  pending[1].wait()


def _concat_body(xt_ref, emba_ref, embb_ref, out_ref):
  out_ref[:NON_CATEG, :] = xt_ref[N_FIELDS:, :]
  ea = emba_ref[...].reshape(GROUP * LATENT_DIM, CONCAT_BLOCK)
  eb = embb_ref[...].reshape(GROUP * LATENT_DIM, CONCAT_BLOCK)
  out_ref[NON_CATEG:NON_CATEG + GROUP * LATENT_DIM, :] = ea
  out_ref[NON_CATEG + GROUP * LATENT_DIM:, :] = (
      eb[:(N_FIELDS - GROUP) * LATENT_DIM, :])


@jax.jit
def kernel(X, raw_emb_tables):
  XT = X.T
  raw_t = raw_emb_tables.transpose(0, 2, 1)
  nprep = BATCH // PREP_BLOCK
  tab2, idxt = pl.pallas_call(
      _prep_body,
      grid=(nprep,),
      in_specs=[
          pl.BlockSpec((DIM, PREP_BLOCK), lambda k: (0, k)),
          pl.BlockSpec((N_FIELDS, LATENT_DIM, NUM_CATEGORIES),
                       lambda k: (0, 0, 0)),
      ],
      out_specs=[
          pl.BlockSpec((N_FIELDS, LATENT_DIM, KPAD // 128, 128),
                       lambda k: (0, 0, 0, 0)),
          pl.BlockSpec((N_FIELDS, PREP_BLOCK // 128, 128),
                       lambda k: (0, k, 0)),
      ],
      out_shape=[
          jax.ShapeDtypeStruct((N_FIELDS, LATENT_DIM, KPAD // 128, 128),
                               jnp.float32),
          jax.ShapeDtypeStruct((N_FIELDS, BATCH // 128, 128), jnp.int32),
      ],
  )(XT, raw_t)

  mesh = plsc.VectorSubcoreMesh(core_axis_name="c", subcore_axis_name="s")
  embat, embbt = pl.kernel(
      _sc_body,
      out_type=[
          jax.ShapeDtypeStruct((GROUP * LATENT_DIM, BATCH // 128, 128),
                               jnp.float32),
          jax.ShapeDtypeStruct((GROUP * LATENT_DIM, BATCH // 128, 128),
                               jnp.float32),
      ],
      mesh=mesh,
      compiler_params=pltpu.CompilerParams(
          use_tc_tiling_on_sc=False, needs_layout_passes=False),
      scratch_types=[
          pltpu.VMEM((LATENT_DIM, KPAD // 128, 128), jnp.float32),
          pltpu.VMEM((LATENT_DIM, KPAD // 128, 128), jnp.float32),
          pltpu.VMEM((TASK_ROWS // 128, 128), jnp.int32),
          pltpu.VMEM((TASK_ROWS // 128, 128), jnp.int32),
          pltpu.VMEM((LATENT_DIM, TASK_ROWS // 128, 128), jnp.float32),
          pltpu.VMEM((LATENT_DIM, TASK_ROWS // 128, 128), jnp.float32),
          pltpu.SemaphoreType.DMA,
          pltpu.SemaphoreType.DMA,
          pltpu.SemaphoreType.DMA,
          pltpu.SemaphoreType.DMA,
      ],
  )(tab2, idxt)

  nblk = BATCH // CONCAT_BLOCK
  outt = pl.pallas_call(
      _concat_body,
      grid=(nblk,),
      in_specs=[
          pl.BlockSpec((DIM, CONCAT_BLOCK), lambda k: (0, k)),
          pl.BlockSpec((GROUP * LATENT_DIM, CONCAT_BLOCK // 128, 128),
                       lambda k: (0, k, 0)),
          pl.BlockSpec((GROUP * LATENT_DIM, CONCAT_BLOCK // 128, 128),
                       lambda k: (0, k, 0)),
      ],
      out_specs=pl.BlockSpec((OUT_COLS, CONCAT_BLOCK), lambda k: (0, k)),
      out_shape=jax.ShapeDtypeStruct((OUT_COLS, BATCH), jnp.float32),
  )(XT, embat, embbt)
  return outt.T
